# baseline (speedup 1.0000x reference)
import functools

import jax
import jax.numpy as jnp
from jax import lax
from jax.experimental import pallas as pl
from jax.experimental.pallas import tpu as pltpu
from jax.experimental.pallas import tpu_sc as plsc

B, D, V, K = 32, 64, 1000000, 256
CHUNK = 16384
NCHUNK = pl.cdiv(V, CHUNK)
VP = NCHUNK * CHUNK
SUB = 128
SPC = CHUNK // SUB
NSUB = VP // SUB
NEG = -3.0e38
GB = 16
NEL = 20480
NV16 = NSUB // 16
_SC_STAGE = 6


def _p1_body(sw_ref, sw2_ref, w_ref, w2_ref, dis_ref, m_ref, t_ref):
    i = pl.program_id(0)
    sw = sw_ref[...]
    wblk = w_ref[...]
    ww = 2.0 * jax.lax.dot_general(
        sw, wblk, (((1,), (0,)), ((), ())),
        preferred_element_type=jnp.float32)
    dis = (sw2_ref[...] - ww) + w2_ref[...]
    col = jax.lax.broadcasted_iota(jnp.int32, (B, CHUNK), 1) + i * CHUNK
    dis = jnp.where(col < V, dis, NEG)
    dis_ref[...] = dis
    mx = jnp.max(dis.reshape(B, SPC, SUB), axis=2)
    m_ref[:, pl.ds(i * SPC, SPC)] = mx

    @pl.when(i == NCHUNK - 1)
    def _bisect():
        m = m_ref[...]
        valid = m > -1.0e38
        lo0 = jnp.min(jnp.where(valid, m, 1.0e30), axis=1, keepdims=True)
        hi0 = jnp.max(m, axis=1, keepdims=True) + 1.0
        def body(_, lohi):
            lo, hi = lohi
            mid = 0.5 * (lo + hi)
            cnt = jnp.sum((m >= mid).astype(jnp.float32), axis=1, keepdims=True)
            take = cnt >= K
            return (jnp.where(take, mid, lo), jnp.where(take, hi, mid))

        lo, hi = jax.lax.fori_loop(0, 48, body, (lo0, hi0))
        t_ref[...] = jnp.broadcast_to(lo, (B, 16))


def _p1(sample_w, sample_w2, weight, w2):
    return pl.pallas_call(
        _p1_body,
        grid=(NCHUNK,),
        in_specs=[
            pl.BlockSpec((B, D), lambda i: (0, 0)),
            pl.BlockSpec((B, 1), lambda i: (0, 0)),
            pl.BlockSpec((D, CHUNK), lambda i: (0, i)),
            pl.BlockSpec((1, CHUNK), lambda i: (0, i)),
        ],
        out_specs=[
            pl.BlockSpec((B, CHUNK), lambda i: (0, i)),
            pl.BlockSpec((B, NSUB), lambda i: (0, 0)),
            pl.BlockSpec((B, 16), lambda i: (0, 0)),
        ],
        out_shape=[
            jax.ShapeDtypeStruct((B, VP), jnp.float32),
            jax.ShapeDtypeStruct((B, NSUB), jnp.float32),
            jax.ShapeDtypeStruct((B, 16), jnp.float32),
        ],
    )(sample_w, sample_w2, weight, w2)


def _iota16():
    return lax.iota(jnp.int32, 16)


def _splat(x, dtype=jnp.int32):
    return jnp.full((16,), x, dtype)


def _key_u32(v):
    kb = plsc.bitcast(v, jnp.int32)
    flipped = jnp.where(kb >= 0, kb ^ jnp.int32(-2147483648), ~kb)
    return plsc.bitcast(flipped, jnp.uint32)


def _sc_body(dis2, m_hbm, t_hbm, w_hbm, x_hbm, mu_hbm, var_hbm,
             score_hbm, idx_hbm,
             m_buf, t_buf, idbuf, gbuf, kbuf, ibuf, selk, seli, outi,
             topkw, densbuf, srow, sem):
    nc = 2
    wid = lax.axis_index("s") * nc + lax.axis_index("c")
    b = wid
    pltpu.sync_copy(m_hbm.at[pl.ds(b * NSUB, NSUB)], m_buf)
    pltpu.sync_copy(t_hbm.at[pl.ds(b * 16, 16)], t_buf)
    tvec = t_buf[...]
    base = _splat(b * NSUB)

    def init_body(i, carry):
        idbuf[pl.ds(i * 16, 16)] = base
        return carry

    lax.fori_loop(0, NV16, init_body, 0)

    def scan_body(i, cnt):
        v = m_buf[pl.ds(i * 16, 16)]
        msk = v >= tvec
        ids = base + _splat(i * 16) + _iota16()
        inc = plsc.cumsum(jnp.where(msk, 1, 0))
        tgt = _splat(cnt) + inc - 1
        plsc.store_scatter(idbuf, [tgt], ids, mask=msk)
        return cnt + jnp.sum(jnp.where(msk, 1, 0))

    n_sub = lax.fori_loop(0, NV16, scan_body, jnp.int32(0))

    def _dummy_out(val):
        lane0 = _iota16() == 0
        plsc.store_scatter(outi, [_splat(0)], _splat(val), mask=lane0)
        pltpu.sync_copy(outi, idx_hbm.at[pl.ds(b * K, K)])
        pltpu.sync_copy(srow.at[pl.ds(5 * D, K)], score_hbm.at[pl.ds(b * K, K)])

    if _SC_STAGE <= 1:
        _dummy_out(n_sub)
        return

    def cd_cond(carry):
        r0, nel = carry
        return r0 < n_sub

    def cd_body(carry):
        r0, nel = carry
        cp = pltpu.make_async_copy(
            dis2.at[idbuf.at[pl.ds(pl.multiple_of(r0, GB), GB)]], gbuf, sem)
        cp.start()
        cp.wait()

        def row_body(r, nel):
            rowvalid = _splat(r0 + r) < _splat(n_sub)
            ida = plsc.load_gather(idbuf, [_splat(r0 + r)])
            colbase = (ida - base) * SUB
            for j in range(SUB // 16):
                v = gbuf[r, pl.ds(j * 16, 16)]
                msk = (v >= tvec) & rowvalid
                gi = colbase + _splat(j * 16) + _iota16()
                key = plsc.bitcast(_key_u32(v), jnp.int32)
                inc = plsc.cumsum(jnp.where(msk, 1, 0))
                tgt = _splat(nel) + inc - 1
                msk = msk & (tgt < NEL)
                plsc.store_scatter(kbuf, [tgt], key, mask=msk)
                plsc.store_scatter(ibuf, [tgt], gi, mask=msk)
                nel = nel + jnp.sum(jnp.where(msk, 1, 0))
            return nel

        nel = lax.fori_loop(0, GB, row_body, nel)
        return (r0 + GB, nel)

    _, nel = lax.while_loop(cd_cond, cd_body, (jnp.int32(0), jnp.int32(0)))
    nv = (nel + 15) // 16
    nel_s = _splat(nel)

    if _SC_STAGE <= 2:
        _dummy_out(nel)
        return

    def bit_body(it, u):
        cand = u | (jnp.uint32(1) << (jnp.uint32(31) - it.astype(jnp.uint32)))
        cand_v = _splat(cand, jnp.uint32)

        def cnt_body(i, acc):
            kv = plsc.bitcast(kbuf[pl.ds(i * 16, 16)], jnp.uint32)
            pos = _splat(i * 16) + _iota16() < nel_s
            m = (kv >= cand_v) & pos
            return acc + plsc.all_reduce_population_count(m)

        acc = lax.fori_loop(0, nv, cnt_body, _splat(0))
        cnt = jnp.max(acc)
        return jnp.where(cnt >= K, cand, u)

    u = lax.fori_loop(0, 32, bit_body, jnp.uint32(0))
    u_v = _splat(u, jnp.uint32)

    if _SC_STAGE <= 3:
        _dummy_out(plsc.bitcast(u, jnp.int32))
        return

    def sel_gt(i, mcnt):
        kv = plsc.bitcast(kbuf[pl.ds(i * 16, 16)], jnp.uint32)
        pos = _splat(i * 16) + _iota16() < nel_s
        m = (kv > u_v) & pos
        inc = plsc.cumsum(jnp.where(m, 1, 0))
        tgt = mcnt + inc - 1
        plsc.store_scatter(selk, [tgt], plsc.bitcast(kv, jnp.int32), mask=m)
        gv = ibuf[pl.ds(i * 16, 16)]
        plsc.store_scatter(seli, [tgt], gv, mask=m)
        return mcnt + plsc.all_reduce_population_count(m)

    mcnt = lax.fori_loop(0, nv, sel_gt, _splat(0))

    def sel_eq(i, mcnt):
        kv = plsc.bitcast(kbuf[pl.ds(i * 16, 16)], jnp.uint32)
        pos = _splat(i * 16) + _iota16() < nel_s
        m = (kv == u_v) & pos
        inc = plsc.cumsum(jnp.where(m, 1, 0))
        tgt = mcnt + inc - 1
        m = m & (tgt < K)
        plsc.store_scatter(selk, [tgt], plsc.bitcast(kv, jnp.int32), mask=m)
        gv = ibuf[pl.ds(i * 16, 16)]
        plsc.store_scatter(seli, [tgt], gv, mask=m)
        return mcnt + plsc.all_reduce_population_count(m)

    lax.fori_loop(0, nv, sel_eq, mcnt)

    def rank_body(i, carry):
        ki = plsc.bitcast(plsc.load_gather(selk, [_splat(i)]), jnp.uint32)
        gii = plsc.load_gather(seli, [_splat(i)])
        i_v = _splat(i)

        def cmp_body(jv, accs):
            agt, aeq = accs
            kv = plsc.bitcast(selk[pl.ds(jv * 16, 16)], jnp.uint32)
            jpos = _splat(jv * 16) + _iota16()
            g = kv > ki
            e = (kv == ki) & (jpos < i_v)
            return (agt + plsc.all_reduce_population_count(g),
                    aeq + plsc.all_reduce_population_count(e))

        agt, aeq = lax.fori_loop(0, K // 16, cmp_body, (_splat(0), _splat(0)))
        rank = agt + aeq
        lane0 = _iota16() == 0
        plsc.store_scatter(outi, [rank], gii, mask=lane0)
        return carry

    lax.fori_loop(0, K, rank_body, 0)

    if _SC_STAGE <= 4:
        pltpu.sync_copy(outi, idx_hbm.at[pl.ds(b * K, K)])
        pltpu.sync_copy(srow.at[pl.ds(5 * D, K)], score_hbm.at[pl.ds(b * K, K)])
        return

    for q in range(K // 16):
        vv = outi[pl.ds(q * 16, 16)]
        ibuf[pl.ds(NEL - K + q * 16, 16)] = vv >> 1
    cp = pltpu.make_async_copy(
        w_hbm.at[ibuf.at[pl.ds(NEL - K, K)]], topkw, sem)
    cp.start()
    cp.wait()
    pltpu.sync_copy(x_hbm.at[pl.ds(b * D, D)], srow.at[pl.ds(0, D)])
    pltpu.sync_copy(mu_hbm.at[pl.ds(b * D, D)], srow.at[pl.ds(D, D)])
    pltpu.sync_copy(var_hbm.at[pl.ds(b * D, D)], srow.at[pl.ds(2 * D, D)])

    if _SC_STAGE <= 5:
        pltpu.sync_copy(outi, idx_hbm.at[pl.ds(b * K, K)])
        pltpu.sync_copy(topkw.at[0].at[pl.ds(0, 16)], score_hbm.at[pl.ds(b * K, 16)])
        pltpu.sync_copy(srow.at[pl.ds(5 * D, K)], score_hbm.at[pl.ds(b * K, K)])
        return

    def dens_body(k, carry):
        vk = plsc.load_gather(outi, [_splat(k)])
        odd = (vk & 1) == 1
        for j in range(D // 16):
            wlo = topkw[k, pl.ds(j * 16, 16)]
            whi = topkw[k, pl.ds(D + j * 16, 16)]
            w = jnp.where(odd, whi, wlo)
            mu_j = srow[pl.ds(D + j * 16, 16)]
            var_j = srow[pl.ds(2 * D + j * 16, 16)]
            dm = w - mu_j
            a = -(dm * dm) / (2.0 * var_j)
            dens = jnp.exp(a)
            densbuf[k, pl.ds(j * 16, 16)] = dens
            s = srow[pl.ds(3 * D + j * 16, 16)]
            mx = srow[pl.ds(4 * D + j * 16, 16)]
            srow[pl.ds(3 * D + j * 16, 16)] = s + dens
            srow[pl.ds(4 * D + j * 16, 16)] = jnp.maximum(mx, dens)
        return carry

    for j in range(D // 16):
        srow[pl.ds(3 * D + j * 16, 16)] = jnp.zeros((16,), jnp.float32)
        srow[pl.ds(4 * D + j * 16, 16)] = jnp.zeros((16,), jnp.float32)
    lax.fori_loop(0, K, dens_body, 0)

    for j in range(D // 16):
        s = srow[pl.ds(3 * D + j * 16, 16)]
        mx = srow[pl.ds(4 * D + j * 16, 16)]
        denom = jnp.maximum(s, 1e-8)
        tau = jnp.minimum((mx / denom) * 0.5, 0.1)
        srow[pl.ds(3 * D + j * 16, 16)] = tau * denom

    def score_body(k, carry):
        vk = plsc.load_gather(outi, [_splat(k)])
        odd = (vk & 1) == 1
        acc = jnp.zeros((16,), jnp.float32)
        for j in range(D // 16):
            wlo = topkw[k, pl.ds(j * 16, 16)]
            whi = topkw[k, pl.ds(D + j * 16, 16)]
            w = jnp.where(odd, whi, wlo)
            dens = densbuf[k, pl.ds(j * 16, 16)]
            thr = srow[pl.ds(3 * D + j * 16, 16)]
            xj = srow[pl.ds(j * 16, 16)]
            acc = acc + jnp.where(dens >= thr, xj * w, 0.0)
        sc = jnp.sum(acc)
        lane0 = _iota16() == 0
        plsc.store_scatter(srow, [_splat(5 * D + k)], _splat(sc, jnp.float32), mask=lane0)
        return carry

    lax.fori_loop(0, K, score_body, 0)
    pltpu.sync_copy(srow.at[pl.ds(5 * D, K)], score_hbm.at[pl.ds(b * K, K)])
    pltpu.sync_copy(outi, idx_hbm.at[pl.ds(b * K, K)])


def _sc_call(dis2, m_arr, t_arr, weight, x, mu, var):
    mesh = plsc.VectorSubcoreMesh(core_axis_name="c", subcore_axis_name="s")
    f = pl.kernel(
        _sc_body,
        mesh=mesh,
        compiler_params=pltpu.CompilerParams(needs_layout_passes=False),
        out_type=[
            jax.ShapeDtypeStruct((B * K,), jnp.float32),
            jax.ShapeDtypeStruct((B * K,), jnp.int32),
        ],
        scratch_types=[
            pltpu.VMEM((NSUB,), jnp.float32),
            pltpu.VMEM((16,), jnp.float32),
            pltpu.VMEM((NSUB,), jnp.int32),
            pltpu.VMEM((GB, SUB), jnp.float32),
            pltpu.VMEM((NEL,), jnp.int32),
            pltpu.VMEM((NEL,), jnp.int32),
            pltpu.VMEM((K,), jnp.int32),
            pltpu.VMEM((K,), jnp.int32),
            pltpu.VMEM((K,), jnp.int32),
            pltpu.VMEM((K, 2 * D), jnp.float32),
            pltpu.VMEM((K, D), jnp.float32),
            pltpu.VMEM((5 * D + K,), jnp.float32),
            pltpu.SemaphoreType.DMA,
        ],
    )
    return f(dis2, m_arr, t_arr, weight, x, mu, var)


def kernel(x, mu, var, labels, weight, bias):
    sample_weight = weight[labels]
    sample_w2 = (sample_weight ** 2).sum(axis=1, keepdims=True)
    w2 = (weight ** 2).sum(axis=1)[None, :]
    dis, m_arr, t_arr = _p1(sample_weight, sample_w2, weight.T, w2)
    dis2 = dis.reshape(B * NSUB, SUB)
    weightr = jnp.concatenate([weight[0::2], weight[1::2]], axis=1)
    score, topk_indice = _sc_call(
        dis2, m_arr.reshape(-1), t_arr.reshape(-1), weightr,
        x.reshape(-1), mu.reshape(-1), var.reshape(-1))
    score = score.reshape(B, K)
    topk_indice = topk_indice.reshape(B, K)
    return score + bias[topk_indice]


# speedup vs baseline: 3.0745x; 3.0745x over previous
import functools

import jax
import jax.numpy as jnp
from jax import lax
from jax.experimental import pallas as pl
from jax.experimental.pallas import tpu as pltpu
from jax.experimental.pallas import tpu_sc as plsc

B, D, V, K = 32, 64, 1000000, 256
CHUNK = 16384
NCHUNK = pl.cdiv(V, CHUNK)
VP = NCHUNK * CHUNK
SUB = 128
SPC = CHUNK // SUB
NSUB = VP // SUB
NEG = -3.0e38
GB = 16
NEL = 20480
NV16 = NSUB // 16
_SC_STAGE = 6


def _p1_body(sw_ref, sw2_ref, w_ref, w2_ref, dis_ref, m_ref, t_ref):
    i = pl.program_id(0)
    sw = sw_ref[...]
    wblk = w_ref[...]
    ww = 2.0 * jax.lax.dot_general(
        sw, wblk, (((1,), (0,)), ((), ())),
        preferred_element_type=jnp.float32)
    dis = (sw2_ref[...] - ww) + w2_ref[...]
    col = jax.lax.broadcasted_iota(jnp.int32, (B, CHUNK), 1) + i * CHUNK
    dis = jnp.where(col < V, dis, NEG)
    dis_ref[...] = dis
    mx = jnp.max(dis.reshape(B, SPC, SUB), axis=2)
    m_ref[:, pl.ds(i * SPC, SPC)] = mx

    @pl.when(i == NCHUNK - 1)
    def _bisect():
        m = m_ref[...]
        valid = m > -1.0e38
        lo0 = jnp.min(jnp.where(valid, m, 1.0e30), axis=1, keepdims=True)
        hi0 = jnp.max(m, axis=1, keepdims=True) + 1.0
        def body(_, lohi):
            lo, hi = lohi
            mid = 0.5 * (lo + hi)
            cnt = jnp.sum((m >= mid).astype(jnp.float32), axis=1, keepdims=True)
            take = cnt >= K
            return (jnp.where(take, mid, lo), jnp.where(take, hi, mid))

        lo, hi = jax.lax.fori_loop(0, 48, body, (lo0, hi0))
        t_ref[...] = jnp.broadcast_to(lo, (B, 16))


def _p1(sample_w, sample_w2, weight, w2):
    return pl.pallas_call(
        _p1_body,
        grid=(NCHUNK,),
        in_specs=[
            pl.BlockSpec((B, D), lambda i: (0, 0)),
            pl.BlockSpec((B, 1), lambda i: (0, 0)),
            pl.BlockSpec((D, CHUNK), lambda i: (0, i)),
            pl.BlockSpec((1, CHUNK), lambda i: (0, i)),
        ],
        out_specs=[
            pl.BlockSpec((B, CHUNK), lambda i: (0, i)),
            pl.BlockSpec((B, NSUB), lambda i: (0, 0)),
            pl.BlockSpec((B, 16), lambda i: (0, 0)),
        ],
        out_shape=[
            jax.ShapeDtypeStruct((B, VP), jnp.float32),
            jax.ShapeDtypeStruct((B, NSUB), jnp.float32),
            jax.ShapeDtypeStruct((B, 16), jnp.float32),
        ],
    )(sample_w, sample_w2, weight, w2)


def _iota16():
    return lax.iota(jnp.int32, 16)


def _splat(x, dtype=jnp.int32):
    return jnp.full((16,), x, dtype)


def _key_u32(v):
    kb = plsc.bitcast(v, jnp.int32)
    flipped = jnp.where(kb >= 0, kb ^ jnp.int32(-2147483648), ~kb)
    return plsc.bitcast(flipped, jnp.uint32)


def _sc_body(dis2, m_hbm, t_hbm, w_hbm, x_hbm, mu_hbm, var_hbm,
             score_hbm, idx_hbm,
             m_buf, t_buf, idbuf, gbuf, kbuf, ibuf, selk, seli, outi,
             topkw, densbuf, srow, sem):
    nc = 2
    wid = lax.axis_index("s") * nc + lax.axis_index("c")
    b = wid
    pltpu.sync_copy(m_hbm.at[pl.ds(b * NSUB, NSUB)], m_buf)
    pltpu.sync_copy(t_hbm.at[pl.ds(b * 16, 16)], t_buf)
    tvec = t_buf[...]
    base = _splat(b * NSUB)

    def init_body(i, carry):
        idbuf[pl.ds(i * 16, 16)] = base
        return carry

    lax.fori_loop(0, NV16, init_body, 0)

    def scan_body(i, cnt):
        v = m_buf[pl.ds(i * 16, 16)]
        msk = v >= tvec
        ids = base + _splat(i * 16) + _iota16()
        inc = plsc.cumsum(jnp.where(msk, 1, 0))
        tgt = _splat(cnt) + inc - 1
        plsc.store_scatter(idbuf, [tgt], ids, mask=msk)
        return cnt + jnp.sum(jnp.where(msk, 1, 0))

    n_sub = lax.fori_loop(0, NV16, scan_body, jnp.int32(0))

    def _dummy_out(val):
        lane0 = _iota16() == 0
        plsc.store_scatter(outi, [_splat(0)], _splat(val), mask=lane0)
        pltpu.sync_copy(outi, idx_hbm.at[pl.ds(b * K, K)])
        pltpu.sync_copy(srow.at[pl.ds(5 * D, K)], score_hbm.at[pl.ds(b * K, K)])

    if _SC_STAGE <= 1:
        _dummy_out(n_sub)
        return

    def cd_cond(carry):
        r0, nel = carry
        return r0 < n_sub

    def cd_body(carry):
        r0, nel = carry
        cp = pltpu.make_async_copy(
            dis2.at[idbuf.at[pl.ds(pl.multiple_of(r0, GB), GB)]], gbuf, sem)
        cp.start()
        cp.wait()

        def row_body(r, nel):
            rowvalid = _splat(r0 + r) < _splat(n_sub)
            ida = plsc.load_gather(idbuf, [_splat(r0 + r)])
            colbase = (ida - base) * SUB
            for j in range(SUB // 16):
                v = gbuf[r, pl.ds(j * 16, 16)]
                msk = (v >= tvec) & rowvalid
                gi = colbase + _splat(j * 16) + _iota16()
                key = plsc.bitcast(_key_u32(v), jnp.int32)
                inc = plsc.cumsum(jnp.where(msk, 1, 0))
                tgt = _splat(nel) + inc - 1
                msk = msk & (tgt < NEL)
                plsc.store_scatter(kbuf, [tgt], key, mask=msk)
                plsc.store_scatter(ibuf, [tgt], gi, mask=msk)
                nel = nel + jnp.sum(jnp.where(msk, 1, 0))
            return nel

        nel = lax.fori_loop(0, GB, row_body, nel)
        return (r0 + GB, nel)

    _, nel = lax.while_loop(cd_cond, cd_body, (jnp.int32(0), jnp.int32(0)))
    nv = (nel + 15) // 16
    nel_s = _splat(nel)

    if _SC_STAGE <= 2:
        _dummy_out(nel)
        return

    def bit_body(it, u):
        cand = u | (jnp.uint32(1) << (jnp.uint32(31) - it.astype(jnp.uint32)))
        cand_v = _splat(cand, jnp.uint32)

        def cnt_body(i, acc):
            kv = plsc.bitcast(kbuf[pl.ds(i * 16, 16)], jnp.uint32)
            pos = _splat(i * 16) + _iota16() < nel_s
            m = (kv >= cand_v) & pos
            return acc + plsc.all_reduce_population_count(m)

        acc = lax.fori_loop(0, nv, cnt_body, _splat(0))
        cnt = jnp.max(acc)
        return jnp.where(cnt >= K, cand, u)

    u = lax.fori_loop(0, 32, bit_body, jnp.uint32(0))
    u_v = _splat(u, jnp.uint32)

    if _SC_STAGE <= 3:
        _dummy_out(plsc.bitcast(u, jnp.int32))
        return

    def sel_gt(i, mcnt):
        kv = plsc.bitcast(kbuf[pl.ds(i * 16, 16)], jnp.uint32)
        pos = _splat(i * 16) + _iota16() < nel_s
        m = (kv > u_v) & pos
        inc = plsc.cumsum(jnp.where(m, 1, 0))
        tgt = mcnt + inc - 1
        plsc.store_scatter(selk, [tgt], plsc.bitcast(kv, jnp.int32), mask=m)
        gv = ibuf[pl.ds(i * 16, 16)]
        plsc.store_scatter(seli, [tgt], gv, mask=m)
        return mcnt + plsc.all_reduce_population_count(m)

    mcnt = lax.fori_loop(0, nv, sel_gt, _splat(0))

    def sel_eq(i, mcnt):
        kv = plsc.bitcast(kbuf[pl.ds(i * 16, 16)], jnp.uint32)
        pos = _splat(i * 16) + _iota16() < nel_s
        m = (kv == u_v) & pos
        inc = plsc.cumsum(jnp.where(m, 1, 0))
        tgt = mcnt + inc - 1
        m = m & (tgt < K)
        plsc.store_scatter(selk, [tgt], plsc.bitcast(kv, jnp.int32), mask=m)
        gv = ibuf[pl.ds(i * 16, 16)]
        plsc.store_scatter(seli, [tgt], gv, mask=m)
        return mcnt + plsc.all_reduce_population_count(m)

    lax.fori_loop(0, nv, sel_eq, mcnt)

    def rank_body(i, carry):
        ki = plsc.bitcast(plsc.load_gather(selk, [_splat(i)]), jnp.uint32)
        gii = plsc.load_gather(seli, [_splat(i)])
        i_v = _splat(i)

        def cmp_body(jv, accs):
            agt, aeq = accs
            kv = plsc.bitcast(selk[pl.ds(jv * 16, 16)], jnp.uint32)
            jpos = _splat(jv * 16) + _iota16()
            g = kv > ki
            e = (kv == ki) & (jpos < i_v)
            return (agt + plsc.all_reduce_population_count(g),
                    aeq + plsc.all_reduce_population_count(e))

        agt, aeq = lax.fori_loop(0, K // 16, cmp_body, (_splat(0), _splat(0)))
        rank = agt + aeq
        lane0 = _iota16() == 0
        plsc.store_scatter(outi, [rank], gii, mask=lane0)
        return carry

    lax.fori_loop(0, K, rank_body, 0)

    if _SC_STAGE <= 4:
        pltpu.sync_copy(outi, idx_hbm.at[pl.ds(b * K, K)])
        pltpu.sync_copy(srow.at[pl.ds(5 * D, K)], score_hbm.at[pl.ds(b * K, K)])
        return

    for q in range(K // 16):
        vv = outi[pl.ds(q * 16, 16)]
        ibuf[pl.ds(NEL - K + q * 16, 16)] = vv >> 1
    cp = pltpu.make_async_copy(
        w_hbm.at[ibuf.at[pl.ds(NEL - K, K)]], topkw, sem)
    cp.start()
    cp.wait()
    pltpu.sync_copy(x_hbm.at[pl.ds(b * D, D)], srow.at[pl.ds(0, D)])
    pltpu.sync_copy(mu_hbm.at[pl.ds(b * D, D)], srow.at[pl.ds(D, D)])
    pltpu.sync_copy(var_hbm.at[pl.ds(b * D, D)], srow.at[pl.ds(2 * D, D)])

    if _SC_STAGE <= 5:
        pltpu.sync_copy(outi, idx_hbm.at[pl.ds(b * K, K)])
        pltpu.sync_copy(topkw.at[0].at[pl.ds(0, 16)], score_hbm.at[pl.ds(b * K, 16)])
        pltpu.sync_copy(srow.at[pl.ds(5 * D, K)], score_hbm.at[pl.ds(b * K, K)])
        return

    def dens_body(k, carry):
        vk = plsc.load_gather(outi, [_splat(k)])
        odd = (vk & 1) == 1
        for j in range(D // 16):
            wlo = topkw[k, pl.ds(j * 16, 16)]
            whi = topkw[k, pl.ds(D + j * 16, 16)]
            w = jnp.where(odd, whi, wlo)
            mu_j = srow[pl.ds(D + j * 16, 16)]
            var_j = srow[pl.ds(2 * D + j * 16, 16)]
            dm = w - mu_j
            a = -(dm * dm) / (2.0 * var_j)
            dens = jnp.exp(a)
            densbuf[k, pl.ds(j * 16, 16)] = dens
            s = srow[pl.ds(3 * D + j * 16, 16)]
            mx = srow[pl.ds(4 * D + j * 16, 16)]
            srow[pl.ds(3 * D + j * 16, 16)] = s + dens
            srow[pl.ds(4 * D + j * 16, 16)] = jnp.maximum(mx, dens)
        return carry

    for j in range(D // 16):
        srow[pl.ds(3 * D + j * 16, 16)] = jnp.zeros((16,), jnp.float32)
        srow[pl.ds(4 * D + j * 16, 16)] = jnp.zeros((16,), jnp.float32)
    lax.fori_loop(0, K, dens_body, 0)

    for j in range(D // 16):
        s = srow[pl.ds(3 * D + j * 16, 16)]
        mx = srow[pl.ds(4 * D + j * 16, 16)]
        denom = jnp.maximum(s, 1e-8)
        tau = jnp.minimum((mx / denom) * 0.5, 0.1)
        srow[pl.ds(3 * D + j * 16, 16)] = tau * denom

    def score_body(k, carry):
        vk = plsc.load_gather(outi, [_splat(k)])
        odd = (vk & 1) == 1
        acc = jnp.zeros((16,), jnp.float32)
        for j in range(D // 16):
            wlo = topkw[k, pl.ds(j * 16, 16)]
            whi = topkw[k, pl.ds(D + j * 16, 16)]
            w = jnp.where(odd, whi, wlo)
            dens = densbuf[k, pl.ds(j * 16, 16)]
            thr = srow[pl.ds(3 * D + j * 16, 16)]
            xj = srow[pl.ds(j * 16, 16)]
            acc = acc + jnp.where(dens >= thr, xj * w, 0.0)
        sc = jnp.sum(acc)
        lane0 = _iota16() == 0
        plsc.store_scatter(srow, [_splat(5 * D + k)], _splat(sc, jnp.float32), mask=lane0)
        return carry

    lax.fori_loop(0, K, score_body, 0)
    pltpu.sync_copy(srow.at[pl.ds(5 * D, K)], score_hbm.at[pl.ds(b * K, K)])
    pltpu.sync_copy(outi, idx_hbm.at[pl.ds(b * K, K)])


def _sc_call(dis2, m_arr, t_arr, weight, x, mu, var):
    mesh = plsc.VectorSubcoreMesh(core_axis_name="c", subcore_axis_name="s")
    f = pl.kernel(
        _sc_body,
        mesh=mesh,
        compiler_params=pltpu.CompilerParams(needs_layout_passes=False),
        out_type=[
            jax.ShapeDtypeStruct((B * K,), jnp.float32),
            jax.ShapeDtypeStruct((B * K,), jnp.int32),
        ],
        scratch_types=[
            pltpu.VMEM((NSUB,), jnp.float32),
            pltpu.VMEM((16,), jnp.float32),
            pltpu.VMEM((NSUB,), jnp.int32),
            pltpu.VMEM((GB, SUB), jnp.float32),
            pltpu.VMEM((NEL,), jnp.int32),
            pltpu.VMEM((NEL,), jnp.int32),
            pltpu.VMEM((K,), jnp.int32),
            pltpu.VMEM((K,), jnp.int32),
            pltpu.VMEM((K,), jnp.int32),
            pltpu.VMEM((K, 2 * D), jnp.float32),
            pltpu.VMEM((K, D), jnp.float32),
            pltpu.VMEM((5 * D + K,), jnp.float32),
            pltpu.SemaphoreType.DMA,
        ],
    )
    return f(dis2, m_arr, t_arr, weight, x, mu, var)


def kernel(x, mu, var, labels, weight, bias):
    sample_weight = weight[labels]
    sample_w2 = (sample_weight ** 2).sum(axis=1, keepdims=True)
    w2 = (weight ** 2).sum(axis=1)[None, :]
    dis, m_arr, t_arr = _p1(sample_weight, sample_w2, weight.T, w2)
    dis2 = dis.reshape(B * NSUB, SUB)
    weightr = weight.reshape(V // 2, 2 * D)
    score, topk_indice = _sc_call(
        dis2, m_arr.reshape(-1), t_arr.reshape(-1), weightr,
        x.reshape(-1), mu.reshape(-1), var.reshape(-1))
    score = score.reshape(B, K)
    topk_indice = topk_indice.reshape(B, K)
    return score + bias[topk_indice]
